# baseline (speedup 1.0000x reference)
import functools

import jax
import jax.numpy as jnp
from jax import lax
from jax.experimental import pallas as pl
from jax.experimental.pallas import tpu as pltpu
from jax.experimental.pallas import tpu_sc as plsc

_NC = 2
_NS = 16
_NW = _NC * _NS
_L = 16
_C = 40
_NB = 4
_NE = 3


def _sc_aggregate(xp, src, dst, edge_attr, N):
    D = edge_attr.shape[1]
    E = edge_attr.shape[0]
    epw = E // _NW
    C = _C
    nchunk = epw // C
    rows_per_tile = (N // _NS) // 8 * 8
    tail_rows = N - rows_per_tile * _NS
    slots = D // _L

    mesh = plsc.VectorSubcoreMesh(core_axis_name="c", subcore_axis_name="s")

    @functools.partial(
        pl.kernel,
        out_type=jax.ShapeDtypeStruct((_NC, N, D), jnp.float32),
        mesh=mesh,
        compiler_params=pltpu.CompilerParams(use_tc_tiling_on_sc=False),
        scratch_types=[
            pltpu.VMEM((epw,), jnp.int32),
            pltpu.VMEM((_NB, C), jnp.int32),
            [pltpu.VMEM((C, D), jnp.float32) for _ in range(_NB)],
            [pltpu.VMEM((C, D // 2), jnp.int32) for _ in range(_NE)],
            pltpu.VMEM_SHARED((N, D), jnp.float32),
            [pltpu.SemaphoreType.DMA for _ in range(_NE)],
            [pltpu.SemaphoreType.DMA for _ in range(_NB)],
            [pltpu.SemaphoreType.DMA for _ in range(_NB)],
            [pltpu.SemaphoreType.DMA for _ in range(_NB)],
        ],
    )
    def agg_kernel(x_hbm, src_hbm, dst_hbm, ea_hbm, out_hbm,
                   src_all, dring, bufs, xgbs, acc_sh, gsem, esem, ssem,
                   isem):
        cid = lax.axis_index("c")
        sid = lax.axis_index("s")
        wid = sid * _NC + cid
        base_w = wid * epw

        def zbody(e, carry):
            for t in range(slots):
                bufs[0][e, pl.ds(t * _L, _L)] = jnp.zeros((_L,), jnp.float32)
            return carry
        lax.fori_loop(0, C, zbody, 0)
        row0 = sid * rows_per_tile
        full, rem = rows_per_tile // C, rows_per_tile % C
        for k in range(full):
            pltpu.sync_copy(bufs[0], acc_sh.at[pl.ds(row0 + k * C, C)])
        if rem:
            pltpu.sync_copy(bufs[0].at[pl.ds(0, rem)],
                            acc_sh.at[pl.ds(row0 + full * C, rem)])
        if tail_rows:
            @pl.when(sid == 0)
            def _zero_tail():
                pltpu.sync_copy(bufs[0].at[pl.ds(0, tail_rows)],
                                acc_sh.at[pl.ds(_NS * rows_per_tile,
                                                tail_rows)])
        plsc.subcore_barrier()

        pltpu.sync_copy(src_hbm.at[pl.ds(base_w, epw)], src_all)

        def fire_dst_idx(j, b):
            pltpu.async_copy(dst_hbm.at[pl.ds(base_w + j * C, C)],
                             dring.at[b], isem[b])

        def wait_dst_idx(b):
            pltpu.make_async_copy(dst_hbm.at[pl.ds(0, C)], dring.at[b],
                                  isem[b]).wait()

        def fire_gathers(j, b, be):
            pltpu.async_copy(x_hbm.at[src_all.at[pl.ds(j * C, C)]],
                             xgbs[be], gsem[be])
            pltpu.async_copy(ea_hbm.at[pl.ds(base_w + j * C, C)],
                             bufs[b], esem[b])

        def wait_gathers(b, be):
            pltpu.make_async_copy(x_hbm.at[src_all.at[pl.ds(0, C)]],
                                  xgbs[be], gsem[be]).wait()
            pltpu.make_async_copy(ea_hbm.at[pl.ds(0, C)], bufs[b],
                                  esem[b]).wait()

        def fire_scatter(b):
            pltpu.async_copy(bufs[b], acc_sh.at[dring.at[b]], ssem[b],
                             add=True)

        def wait_scatter(b):
            pltpu.make_async_copy(bufs[b], acc_sh.at[dring.at[b]],
                                  ssem[b]).wait()

        def compute(b, be):
            buf, xgb = bufs[b], xgbs[be]

            @plsc.parallel_loop(0, C, unroll=4)
            def ebody(e):
                for blk in range(D // (2 * _L)):
                    wv = xgb[e, pl.ds(blk * _L, _L)]
                    lo = lax.bitcast_convert_type(wv * jnp.int32(65536),
                                                  jnp.float32)
                    hi = lax.bitcast_convert_type(
                        jnp.bitwise_and(wv, jnp.int32(-65536)), jnp.float32)
                    sl = pl.ds(blk * 2 * _L, _L)
                    sh = pl.ds(blk * 2 * _L + _L, _L)
                    buf[e, sl] = jnp.maximum(buf[e, sl] + lo, 0.0)
                    buf[e, sh] = jnp.maximum(buf[e, sh] + hi, 0.0)

        def step(j, b, be, fire_next, guard_drain):
            wait_gathers(b, be)
            b2 = (b + 2) % _NB

            def drain():
                wait_scatter(b2)
            if guard_drain:
                pl.when(j >= 2)(drain)
            else:
                drain()
            if fire_next:
                fire_dst_idx(j + 2, b2)
                fire_gathers(j + 2, b2, (be + 2) % _NE)
            compute(b, be)
            wait_dst_idx(b)
            fire_scatter(b)

        fire_dst_idx(0, 0)
        fire_gathers(0, 0, 0)
        fire_dst_idx(1, 1)
        fire_gathers(1, 1, 1)

        def twelve_body(jj, carry):
            for u in range(12):
                step(12 * jj + u, u % _NB, u % _NE, True, u < 2)
            return carry
        ntw = (nchunk - 2) // 12
        lax.fori_loop(0, ntw, twelve_body, 0)
        for j in range(12 * ntw, nchunk):
            step(j, j % _NB, j % _NE, j + 2 < nchunk, False)
        for j in range(nchunk - 2, nchunk):
            wait_scatter(j % _NB)

        plsc.subcore_barrier()
        pltpu.sync_copy(acc_sh.at[pl.ds(row0, rows_per_tile)],
                        out_hbm.at[cid, pl.ds(row0, rows_per_tile)])
        if tail_rows:
            @pl.when(sid == 0)
            def _dump_tail():
                pltpu.sync_copy(
                    acc_sh.at[pl.ds(_NS * rows_per_tile, tail_rows)],
                    out_hbm.at[cid, pl.ds(_NS * rows_per_tile, tail_rows)])

    return agg_kernel(xp, src, dst, edge_attr)


def _tc_body(x_ref, p_ref, w1_ref, b1_ref, w2_ref, b2_ref, g_ref, bt_ref,
             o_ref):
    z = x_ref[...] + p_ref[0] + p_ref[1]
    h1 = lax.dot_general(z, w1_ref[...], (((1,), (1,)), ((), ())),
                         preferred_element_type=jnp.float32)
    h1 = jnp.maximum(h1 + b1_ref[...], 0.0)
    h = lax.dot_general(h1, w2_ref[...], (((1,), (1,)), ((), ())),
                        preferred_element_type=jnp.float32)
    h = h + b2_ref[...]
    mean = jnp.mean(h, axis=0, keepdims=True)
    var = jnp.mean((h - mean) ** 2, axis=0, keepdims=True)
    hn = (h - mean) * lax.rsqrt(var + 1e-5)
    o_ref[...] = jnp.maximum(hn * g_ref[...] + bt_ref[...], 0.0)


def kernel(x, edge_index, edge_attr, W1, b1, W2, b2, gamma, beta):
    N, D = x.shape
    src = edge_index[0]
    dst = edge_index[1]
    u = lax.bitcast_convert_type(x.astype(jnp.bfloat16),
                                 jnp.uint16).astype(jnp.uint32)
    ur = u.reshape(N, D // 32, 2, 16)
    words = ur[:, :, 0, :] | (ur[:, :, 1, :] << 16)
    xp = lax.bitcast_convert_type(words.reshape(N, D // 2), jnp.int32)
    parts = _sc_aggregate(xp, src, dst, edge_attr, N)
    out = pl.pallas_call(
        _tc_body,
        out_shape=jax.ShapeDtypeStruct((N, D), jnp.float32),
    )(x, parts, W1, b1.reshape(1, D), W2, b2.reshape(1, D),
      gamma.reshape(1, D), beta.reshape(1, D))
    return out


# speedup vs baseline: 1.0865x; 1.0865x over previous
import functools

import jax
import jax.numpy as jnp
from jax import lax
from jax.experimental import pallas as pl
from jax.experimental.pallas import tpu as pltpu
from jax.experimental.pallas import tpu_sc as plsc

_NC = 2
_NS = 16
_NW = _NC * _NS
_L = 16
_C = 40
_NB = 5
_NE = 4


def _sc_aggregate(xp, src, dst, edge_attr, N):
    D = edge_attr.shape[1]
    E = edge_attr.shape[0]
    epw = E // _NW
    C = _C
    nchunk = epw // C
    rows_per_tile = (N // _NS) // 8 * 8
    tail_rows = N - rows_per_tile * _NS
    slots = D // _L

    mesh = plsc.VectorSubcoreMesh(core_axis_name="c", subcore_axis_name="s")

    @functools.partial(
        pl.kernel,
        out_type=jax.ShapeDtypeStruct((_NC, N, D), jnp.float32),
        mesh=mesh,
        compiler_params=pltpu.CompilerParams(use_tc_tiling_on_sc=False),
        scratch_types=[
            pltpu.VMEM((epw,), jnp.int32),
            pltpu.VMEM((_NB, C), jnp.int32),
            [pltpu.VMEM((C, D), jnp.float32) for _ in range(_NB)],
            [pltpu.VMEM((C, D // 2), jnp.int32) for _ in range(_NE)],
            pltpu.VMEM_SHARED((N, D), jnp.float32),
            [pltpu.SemaphoreType.DMA for _ in range(_NE)],
            [pltpu.SemaphoreType.DMA for _ in range(_NB)],
            [pltpu.SemaphoreType.DMA for _ in range(_NB)],
            [pltpu.SemaphoreType.DMA for _ in range(_NB)],
        ],
    )
    def agg_kernel(x_hbm, src_hbm, dst_hbm, ea_hbm, out_hbm,
                   src_all, dring, bufs, xgbs, acc_sh, gsem, esem, ssem,
                   isem):
        cid = lax.axis_index("c")
        sid = lax.axis_index("s")
        wid = sid * _NC + cid
        base_w = wid * epw

        def zbody(e, carry):
            for t in range(slots):
                bufs[0][e, pl.ds(t * _L, _L)] = jnp.zeros((_L,), jnp.float32)
            return carry
        lax.fori_loop(0, C, zbody, 0)
        row0 = sid * rows_per_tile
        full, rem = rows_per_tile // C, rows_per_tile % C
        for k in range(full):
            pltpu.sync_copy(bufs[0], acc_sh.at[pl.ds(row0 + k * C, C)])
        if rem:
            pltpu.sync_copy(bufs[0].at[pl.ds(0, rem)],
                            acc_sh.at[pl.ds(row0 + full * C, rem)])
        if tail_rows:
            @pl.when(sid == 0)
            def _zero_tail():
                pltpu.sync_copy(bufs[0].at[pl.ds(0, tail_rows)],
                                acc_sh.at[pl.ds(_NS * rows_per_tile,
                                                tail_rows)])
        plsc.subcore_barrier()

        pltpu.sync_copy(src_hbm.at[pl.ds(base_w, epw)], src_all)

        def fire_dst_idx(j, b):
            pltpu.async_copy(dst_hbm.at[pl.ds(base_w + j * C, C)],
                             dring.at[b], isem[b])

        def wait_dst_idx(b):
            pltpu.make_async_copy(dst_hbm.at[pl.ds(0, C)], dring.at[b],
                                  isem[b]).wait()

        def fire_gathers(j, b, be):
            pltpu.async_copy(x_hbm.at[src_all.at[pl.ds(j * C, C)]],
                             xgbs[be], gsem[be])
            pltpu.async_copy(ea_hbm.at[pl.ds(base_w + j * C, C)],
                             bufs[b], esem[b])

        def wait_gathers(b, be):
            pltpu.make_async_copy(x_hbm.at[src_all.at[pl.ds(0, C)]],
                                  xgbs[be], gsem[be]).wait()
            pltpu.make_async_copy(ea_hbm.at[pl.ds(0, C)], bufs[b],
                                  esem[b]).wait()

        def fire_scatter(b):
            pltpu.async_copy(bufs[b], acc_sh.at[dring.at[b]], ssem[b],
                             add=True)

        def wait_scatter(b):
            pltpu.make_async_copy(bufs[b], acc_sh.at[dring.at[b]],
                                  ssem[b]).wait()

        def compute(b, be):
            buf, xgb = bufs[b], xgbs[be]

            @plsc.parallel_loop(0, C, unroll=4)
            def ebody(e):
                for blk in range(D // (2 * _L)):
                    wv = xgb[e, pl.ds(blk * _L, _L)]
                    lo = lax.bitcast_convert_type(wv * jnp.int32(65536),
                                                  jnp.float32)
                    hi = lax.bitcast_convert_type(
                        jnp.bitwise_and(wv, jnp.int32(-65536)), jnp.float32)
                    sl = pl.ds(blk * 2 * _L, _L)
                    sh = pl.ds(blk * 2 * _L + _L, _L)
                    buf[e, sl] = jnp.maximum(buf[e, sl] + lo, 0.0)
                    buf[e, sh] = jnp.maximum(buf[e, sh] + hi, 0.0)

        def step(j, b, be, fire_next, guard_drain):
            wait_gathers(b, be)
            b3 = (b + 3) % _NB

            def drain():
                wait_scatter(b3)
            if guard_drain:
                pl.when(j >= 2)(drain)
            else:
                drain()
            if fire_next:
                fire_dst_idx(j + 3, b3)
                fire_gathers(j + 3, b3, (be + 3) % _NE)
            compute(b, be)
            wait_dst_idx(b)
            fire_scatter(b)

        for pj in range(3):
            fire_dst_idx(pj, pj)
            fire_gathers(pj, pj, pj)

        def twenty_body(jj, carry):
            for u in range(20):
                step(20 * jj + u, u % _NB, u % _NE, True, u < 2)
            return carry
        ntw = (nchunk - 3) // 20
        lax.fori_loop(0, ntw, twenty_body, 0)
        for j in range(20 * ntw, nchunk):
            step(j, j % _NB, j % _NE, j + 3 < nchunk, False)
        for j in range(nchunk - 2, nchunk):
            wait_scatter(j % _NB)

        plsc.subcore_barrier()
        pltpu.sync_copy(acc_sh.at[pl.ds(row0, rows_per_tile)],
                        out_hbm.at[cid, pl.ds(row0, rows_per_tile)])
        if tail_rows:
            @pl.when(sid == 0)
            def _dump_tail():
                pltpu.sync_copy(
                    acc_sh.at[pl.ds(_NS * rows_per_tile, tail_rows)],
                    out_hbm.at[cid, pl.ds(_NS * rows_per_tile, tail_rows)])

    return agg_kernel(xp, src, dst, edge_attr)


def _tc_body(x_ref, p_ref, w1_ref, b1_ref, w2_ref, b2_ref, g_ref, bt_ref,
             o_ref):
    z = x_ref[...] + p_ref[0] + p_ref[1]
    h1 = lax.dot_general(z, w1_ref[...], (((1,), (1,)), ((), ())),
                         preferred_element_type=jnp.float32)
    h1 = jnp.maximum(h1 + b1_ref[...], 0.0)
    h = lax.dot_general(h1, w2_ref[...], (((1,), (1,)), ((), ())),
                        preferred_element_type=jnp.float32)
    h = h + b2_ref[...]
    mean = jnp.mean(h, axis=0, keepdims=True)
    var = jnp.mean((h - mean) ** 2, axis=0, keepdims=True)
    hn = (h - mean) * lax.rsqrt(var + 1e-5)
    o_ref[...] = jnp.maximum(hn * g_ref[...] + bt_ref[...], 0.0)


def kernel(x, edge_index, edge_attr, W1, b1, W2, b2, gamma, beta):
    N, D = x.shape
    src = edge_index[0]
    dst = edge_index[1]
    u = lax.bitcast_convert_type(x.astype(jnp.bfloat16),
                                 jnp.uint16).astype(jnp.uint32)
    ur = u.reshape(N, D // 32, 2, 16)
    words = ur[:, :, 0, :] | (ur[:, :, 1, :] << 16)
    xp = lax.bitcast_convert_type(words.reshape(N, D // 2), jnp.int32)
    parts = _sc_aggregate(xp, src, dst, edge_attr, N)
    out = pl.pallas_call(
        _tc_body,
        out_shape=jax.ShapeDtypeStruct((N, D), jnp.float32),
    )(x, parts, W1, b1.reshape(1, D), W2, b2.reshape(1, D),
      gamma.reshape(1, D), beta.reshape(1, D))
    return out
